# Initial kernel scaffold; baseline (speedup 1.0000x reference)
import jax
import jax.numpy as jnp
from jax.experimental import pallas as pl


def kernel(x, edge_index, batch, params):
    raise NotImplementedError("write your pallas kernel here")


# speedup vs baseline: 3.2539x; 3.2539x over previous
import functools

import jax
import jax.numpy as jnp
from jax import lax
from jax.experimental import pallas as pl
from jax.experimental.pallas import tpu as pltpu
from jax.experimental.pallas import tpu_sc as plsc

F = 128
EB = 80
NSUB = 16
NCORE = 2
NW = NSUB * NCORE
NGRAPH = 64


@functools.lru_cache(maxsize=None)
def _dense_fn(n, din, ncol):
    br = 1000
    nr = n // br

    def body(x_ref, w_ref, b_ref, o_ref):
        o_ref[...] = jnp.dot(x_ref[...], w_ref[...],
                             preferred_element_type=jnp.float32) + b_ref[...]

    return pl.pallas_call(
        body,
        grid=(nr, ncol),
        in_specs=[
            pl.BlockSpec((br, din), lambda i, j: (i, 0)),
            pl.BlockSpec((din, F), lambda i, j: (0, j)),
            pl.BlockSpec((1, F), lambda i, j: (0, j)),
        ],
        out_specs=pl.BlockSpec((None, br, F), lambda i, j: (j, i, 0)),
        out_shape=jax.ShapeDtypeStruct((ncol, n, F), jnp.float32),
    )


@functools.lru_cache(maxsize=None)
def _edge_fn(nc, n, e):
    ew = e // NW
    nb = ew // EB
    np_ = -(-n // 2048) * 2048
    rps = np_ // NSUB
    dr = 128
    nd = rps // dr
    mesh = plsc.VectorSubcoreMesh(core_axis_name="c", subcore_axis_name="s")

    @functools.partial(
        pl.kernel,
        mesh=mesh,
        out_type=jax.ShapeDtypeStruct((NCORE, nc, np_, F), jnp.float32),
        scratch_types=[
            pltpu.VMEM((EB,), jnp.int32),
            pltpu.VMEM((EB,), jnp.int32),
            pltpu.VMEM((EB,), jnp.int32),
            pltpu.VMEM((EB,), jnp.int32),
            pltpu.VMEM((EB,), jnp.int32),
            pltpu.VMEM((EB, F), jnp.float32),
            pltpu.VMEM((EB, F), jnp.float32),
            pltpu.VMEM((EB, F), jnp.float32),
            pltpu.VMEM((dr, F), jnp.float32),
            pltpu.VMEM_SHARED((np_, F), jnp.float32),
            pltpu.SemaphoreType.DMA,
        ],
    )
    def ek(kqv_hbm, src_hbm, dst_hbm, out_hbm,
           sbuf, dbuf, ik, iq, iv, kb, qb, vb, zb, agg, sem):
        cid = lax.axis_index("c")
        sid = lax.axis_index("s")
        wid = sid * NCORE + cid
        base = wid * ew
        row0 = sid * rps

        def zrow(r, carry):
            for g in range(F // 16):
                zb[r, pl.ds(g * 16, 16)] = jnp.zeros((16,), jnp.float32)
            return carry

        lax.fori_loop(0, dr, zrow, 0)

        for ck in range(nc):
            koff = ck * n
            qoff = (nc + ck) * n
            voff = (2 * nc + ck) * n

            for t in range(nd):
                pltpu.sync_copy(zb, agg.at[pl.ds(row0 + t * dr, dr)])
            plsc.subcore_barrier()

            def blk(b, carry):
                e0 = base + b * EB
                pltpu.sync_copy(src_hbm.at[pl.ds(e0, EB)], sbuf)
                pltpu.sync_copy(dst_hbm.at[pl.ds(e0, EB)], dbuf)
                for g in range(EB // 16):
                    sl = pl.ds(g * 16, 16)
                    s16 = sbuf[sl]
                    d16 = dbuf[sl]
                    ik[sl] = d16 + koff
                    iq[sl] = s16 + qoff
                    iv[sl] = s16 + voff
                c1 = pltpu.async_copy(kqv_hbm.at[ik], kb, sem)
                c2 = pltpu.async_copy(kqv_hbm.at[iq], qb, sem)
                c3 = pltpu.async_copy(kqv_hbm.at[iv], vb, sem)
                c1.wait()
                c2.wait()
                c3.wait()

                def edge(i, c):
                    for g in range(F // 16):
                        sl = pl.ds(g * 16, 16)
                        d = jnp.exp(kb[i, sl] + qb[i, sl]) + 1.0
                        vb[i, sl] = vb[i, sl] / d
                    return c

                lax.fori_loop(0, EB, edge, 0)
                pltpu.sync_copy(vb, agg.at[dbuf], add=True)
                return carry

            lax.fori_loop(0, nb, blk, 0)
            plsc.subcore_barrier()

            for t in range(nd):
                sl = pl.ds(row0 + t * dr, dr)
                pltpu.sync_copy(agg.at[sl], out_hbm.at[cid, ck].at[sl])
            plsc.subcore_barrier()

    return ek


@functools.lru_cache(maxsize=None)
def _post_fn(nc, n, np_):
    def body(p_ref, kqv_ref, g_ref, b_ref, o_ref):
        h = jnp.maximum(p_ref[0, :n] + p_ref[1, :n] + kqv_ref[...], 0.0)
        m = jnp.mean(h, axis=0, keepdims=True)
        d = h - m
        var = jnp.mean(d * d, axis=0, keepdims=True)
        o_ref[...] = g_ref[...] * d / jnp.sqrt(var + 1e-5) + b_ref[...]

    return pl.pallas_call(
        body,
        grid=(nc,),
        in_specs=[
            pl.BlockSpec((NCORE, None, np_, F), lambda j: (0, j, 0, 0)),
            pl.BlockSpec((None, n, F), lambda j: (3 * nc + j, 0, 0)),
            pl.BlockSpec((None, 1, F), lambda j: (j, 0, 0)),
            pl.BlockSpec((None, 1, F), lambda j: (j, 0, 0)),
        ],
        out_specs=pl.BlockSpec((n, F), lambda j: (0, j)),
        out_shape=jax.ShapeDtypeStruct((n, nc * F), jnp.float32),
    )


@functools.lru_cache(maxsize=None)
def _pool_fn(n):
    br = 1000
    nr = n // br

    def body(b_ref, h_ref, o_ref):
        @pl.when(pl.program_id(0) == 0)
        def _():
            o_ref[...] = jnp.zeros_like(o_ref)

        oh = (b_ref[...] == lax.broadcasted_iota(jnp.int32, (1, NGRAPH), 1)
              ).astype(jnp.float32)
        o_ref[...] += lax.dot_general(
            oh, h_ref[...], (((0,), (0,)), ((), ())),
            precision=lax.Precision.HIGHEST,
            preferred_element_type=jnp.float32)

    return pl.pallas_call(
        body,
        grid=(nr,),
        in_specs=[
            pl.BlockSpec((br, 1), lambda i: (i, 0)),
            pl.BlockSpec((br, F), lambda i: (i, 0)),
        ],
        out_specs=pl.BlockSpec((NGRAPH, F), lambda i: (0, 0)),
        out_shape=jax.ShapeDtypeStruct((NGRAPH, F), jnp.float32),
    )


def kernel(x, edge_index, batch, params):
    n = x.shape[0]
    e = edge_index.shape[1]
    src = edge_index[0]
    dst = edge_index[1]

    h = x
    for i, p in enumerate(params['convs']):
        din, dout = p['Wk'].shape
        nc = dout // F
        wall = jnp.concatenate([-p['Wk'], -p['Wq'], p['Wv'], p['Ws']], axis=1)
        ball = jnp.concatenate([-p['bk'], -p['bq'], p['bv'], p['bs']])[None, :]
        kqv = _dense_fn(n, din, 4 * nc)(h, wall, ball)
        parts = _edge_fn(nc, n, e)(kqv.reshape(4 * nc * n, F), src, dst)
        np_ = parts.shape[2]
        g2 = params['bn'][i]['g'].reshape(nc, 1, F)
        b2 = params['bn'][i]['b'].reshape(nc, 1, F)
        h = _post_fn(nc, n, np_)(parts, kqv, g2, b2)

    return _pool_fn(n)(batch.reshape(n, 1), h)


# speedup vs baseline: 5.2428x; 1.6112x over previous
import functools

import jax
import jax.numpy as jnp
from jax import lax
from jax.experimental import pallas as pl
from jax.experimental.pallas import tpu as pltpu
from jax.experimental.pallas import tpu_sc as plsc

F = 128
EB = 48
NSUB = 16
NCORE = 2
NW = NSUB * NCORE
NGRAPH = 64


def _edge_geometry(e):
    nb = -(-e // (NW * EB))
    if nb % 2 == 0:
        nb += 1
    ew = nb * EB
    return nb, ew, NW * ew - e


@functools.lru_cache(maxsize=None)
def _dense_fn(n, din, ncol):
    br = 1000
    nr = n // br

    def body(x_ref, w_ref, b_ref, o_ref):
        o_ref[...] = jnp.dot(x_ref[...], w_ref[...],
                             preferred_element_type=jnp.float32) + b_ref[...]

    return pl.pallas_call(
        body,
        grid=(nr, ncol),
        in_specs=[
            pl.BlockSpec((br, din), lambda i, j: (i, 0)),
            pl.BlockSpec((din, F), lambda i, j: (0, j)),
            pl.BlockSpec((1, F), lambda i, j: (0, j)),
        ],
        out_specs=pl.BlockSpec((None, br, F), lambda i, j: (j, i, 0)),
        out_shape=jax.ShapeDtypeStruct((ncol, n, F), jnp.float32),
    )


@functools.lru_cache(maxsize=None)
def _edge_fn(nc, n, e):
    nb, ew, _ = _edge_geometry(e)
    np_ = -(-n // 128) * 128
    rps = np_ // NSUB
    mesh = plsc.VectorSubcoreMesh(core_axis_name="c", subcore_axis_name="s")

    @functools.partial(
        pl.kernel,
        mesh=mesh,
        out_type=jax.ShapeDtypeStruct((NCORE, nc, np_, F), jnp.float32),
        scratch_types=[
            [pltpu.VMEM((2, EB), jnp.int32) for _ in range(2)],
            [pltpu.VMEM((2 * EB,), jnp.int32) for _ in range(2)],
            [pltpu.VMEM((EB,), jnp.int32) for _ in range(2)],
            [pltpu.VMEM((EB,), jnp.int32) for _ in range(2)],
            [pltpu.VMEM((2 * EB, F), jnp.float32) for _ in range(2)],
            [pltpu.VMEM((EB, F), jnp.float32) for _ in range(2)],
            pltpu.VMEM_SHARED((np_, F), jnp.float32),
            [pltpu.SemaphoreType.DMA for _ in range(2)],
            [pltpu.SemaphoreType.DMA for _ in range(2)],
        ],
    )
    def ek(kqv_hbm, src_hbm, dst_hbm, zrows_hbm, out_hbm,
           sd, ikq, iv, db, kqb, vb, agg, semi, semg):
        cid = lax.axis_index("c")
        sid = lax.axis_index("s")
        wid = sid * NCORE + cid
        base = wid * ew
        row0 = sid * rps
        rows = pl.ds(row0, rps)

        def stage_i(b, p):
            e0 = base + jnp.minimum(b, nb - 1) * EB
            pltpu.async_copy(src_hbm.at[pl.ds(e0, EB)], sd[p].at[0], semi[p])
            pltpu.async_copy(dst_hbm.at[pl.ds(e0, EB)], sd[p].at[1], semi[p])

        def stage_g(b, p, koff, qoff, voff):
            e0 = base + b * EB
            pltpu.make_async_copy(src_hbm.at[pl.ds(e0, EB)], sd[p].at[0],
                                  semi[p]).wait()
            pltpu.make_async_copy(dst_hbm.at[pl.ds(e0, EB)], sd[p].at[1],
                                  semi[p]).wait()
            for g in range(EB // 16):
                gd = pl.ds(g * 16, 16)
                s16 = sd[p][0, gd]
                d16 = sd[p][1, gd]
                ikq[p][gd] = d16 + koff
                ikq[p][pl.ds(EB + g * 16, 16)] = s16 + qoff
                iv[p][gd] = s16 + voff
                db[p][gd] = d16
            pltpu.async_copy(kqv_hbm.at[ikq[p]], kqb[p], semg[p])
            pltpu.async_copy(kqv_hbm.at[iv[p]], vb[p], semg[p])

        def finish(p):
            pltpu.make_async_copy(kqv_hbm.at[ikq[p]], kqb[p], semg[p]).wait()
            pltpu.make_async_copy(kqv_hbm.at[iv[p]], vb[p], semg[p]).wait()

            def edge(i, c):
                for g in range(F // 16):
                    sl = pl.ds(g * 16, 16)
                    d = jnp.exp(kqb[p][i, sl] + kqb[p][EB + i, sl]) + 1.0
                    vb[p][i, sl] = vb[p][i, sl] / d
                return c

            lax.fori_loop(0, EB, edge, 0)
            pltpu.sync_copy(vb[p], agg.at[db[p]], add=True)

        for ck in range(nc):
            koff = ck * n
            qoff = (nc + ck) * n
            voff = (2 * nc + ck) * n

            pltpu.sync_copy(zrows_hbm, agg.at[rows])
            plsc.subcore_barrier()

            stage_i(0, 0)
            stage_i(1, 1)
            stage_g(0, 0, koff, qoff, voff)

            def pair(j, carry):
                b0 = 2 * j
                stage_i(b0 + 2, 0)
                stage_g(b0 + 1, 1, koff, qoff, voff)
                finish(0)
                stage_i(b0 + 3, 1)
                stage_g(b0 + 2, 0, koff, qoff, voff)
                finish(1)
                return carry

            lax.fori_loop(0, (nb - 1) // 2, pair, 0)
            finish(0)
            pltpu.make_async_copy(src_hbm.at[pl.ds(base, EB)],
                                  sd[1].at[0], semi[1]).wait()
            pltpu.make_async_copy(dst_hbm.at[pl.ds(base, EB)],
                                  sd[1].at[1], semi[1]).wait()
            plsc.subcore_barrier()

            pltpu.sync_copy(agg.at[rows], out_hbm.at[cid, ck].at[rows])
            plsc.subcore_barrier()

    return ek


@functools.lru_cache(maxsize=None)
def _post_fn(nc, n, np_):
    def body(p_ref, kqv_ref, g_ref, b_ref, o_ref):
        h = jnp.maximum(p_ref[0, :n] + p_ref[1, :n] + kqv_ref[...], 0.0)
        m = jnp.mean(h, axis=0, keepdims=True)
        d = h - m
        var = jnp.mean(d * d, axis=0, keepdims=True)
        o_ref[...] = g_ref[...] * d / jnp.sqrt(var + 1e-5) + b_ref[...]

    return pl.pallas_call(
        body,
        grid=(nc,),
        in_specs=[
            pl.BlockSpec((NCORE, None, np_, F), lambda j: (0, j, 0, 0)),
            pl.BlockSpec((None, n, F), lambda j: (3 * nc + j, 0, 0)),
            pl.BlockSpec((None, 1, F), lambda j: (j, 0, 0)),
            pl.BlockSpec((None, 1, F), lambda j: (j, 0, 0)),
        ],
        out_specs=pl.BlockSpec((n, F), lambda j: (0, j)),
        out_shape=jax.ShapeDtypeStruct((n, nc * F), jnp.float32),
    )


@functools.lru_cache(maxsize=None)
def _pool_fn(n):
    br = 1000
    nr = n // br

    def body(b_ref, h_ref, o_ref):
        @pl.when(pl.program_id(0) == 0)
        def _():
            o_ref[...] = jnp.zeros_like(o_ref)

        oh = (b_ref[...] == lax.broadcasted_iota(jnp.int32, (1, NGRAPH), 1)
              ).astype(jnp.float32)
        o_ref[...] += lax.dot_general(
            oh, h_ref[...], (((0,), (0,)), ((), ())),
            precision=lax.Precision.HIGHEST,
            preferred_element_type=jnp.float32)

    return pl.pallas_call(
        body,
        grid=(nr,),
        in_specs=[
            pl.BlockSpec((br, 1), lambda i: (i, 0)),
            pl.BlockSpec((br, F), lambda i: (i, 0)),
        ],
        out_specs=pl.BlockSpec((NGRAPH, F), lambda i: (0, 0)),
        out_shape=jax.ShapeDtypeStruct((NGRAPH, F), jnp.float32),
    )


def kernel(x, edge_index, batch, params):
    n = x.shape[0]
    e = edge_index.shape[1]
    _, _, pad = _edge_geometry(e)
    src = jnp.concatenate([edge_index[0], jnp.zeros((pad,), jnp.int32)])
    dst = jnp.concatenate([edge_index[1], jnp.full((pad,), n, jnp.int32)])
    np_ = -(-n // 128) * 128
    zrows = jnp.zeros((np_ // NSUB, F), jnp.float32)

    h = x
    for i, p in enumerate(params['convs']):
        din, dout = p['Wk'].shape
        nc = dout // F
        wall = jnp.concatenate([-p['Wk'], -p['Wq'], p['Wv'], p['Ws']], axis=1)
        ball = jnp.concatenate([-p['bk'], -p['bq'], p['bv'], p['bs']])[None, :]
        kqv = _dense_fn(n, din, 4 * nc)(h, wall, ball)
        parts = _edge_fn(nc, n, e)(kqv.reshape(4 * nc * n, F), src, dst, zrows)
        g2 = params['bn'][i]['g'].reshape(nc, 1, F)
        b2 = params['bn'][i]['b'].reshape(nc, 1, F)
        h = _post_fn(nc, n, parts.shape[2])(parts, kqv, g2, b2)

    return _pool_fn(n)(batch.reshape(n, 1), h)


# speedup vs baseline: 5.2546x; 1.0022x over previous
import functools

import jax
import jax.numpy as jnp
from jax import lax
from jax.experimental import pallas as pl
from jax.experimental.pallas import tpu as pltpu
from jax.experimental.pallas import tpu_sc as plsc

F = 128
EB = 48
NSUB = 16
NCORE = 2
NW = NSUB * NCORE
NGRAPH = 64


def _edge_geometry(e):
    nb = -(-e // (NW * EB))
    if nb % 2 == 0:
        nb += 1
    ew = nb * EB
    return nb, ew, NW * ew - e


@functools.lru_cache(maxsize=None)
def _dense_fn(n, din, ncol):
    br = 1000
    nr = n // br

    def body(x_ref, w_ref, b_ref, o_ref):
        o_ref[...] = jnp.dot(x_ref[...], w_ref[...],
                             preferred_element_type=jnp.float32) + b_ref[...]

    return pl.pallas_call(
        body,
        grid=(nr, ncol),
        in_specs=[
            pl.BlockSpec((br, din), lambda i, j: (i, 0)),
            pl.BlockSpec((din, F), lambda i, j: (0, j)),
            pl.BlockSpec((1, F), lambda i, j: (0, j)),
        ],
        out_specs=pl.BlockSpec((None, br, F), lambda i, j: (j, i, 0)),
        out_shape=jax.ShapeDtypeStruct((ncol, n, F), jnp.float32),
    )


@functools.lru_cache(maxsize=None)
def _edge_fn(nc, n, e):
    nb, ew, _ = _edge_geometry(e)
    np_ = -(-n // 128) * 128
    rps = np_ // NSUB
    mesh = plsc.VectorSubcoreMesh(core_axis_name="c", subcore_axis_name="s")

    @functools.partial(
        pl.kernel,
        mesh=mesh,
        out_type=jax.ShapeDtypeStruct((NCORE, nc, np_, F), jnp.float32),
        scratch_types=[
            [pltpu.VMEM((2, EB), jnp.int32) for _ in range(2)],
            [pltpu.VMEM((2 * EB,), jnp.int32) for _ in range(2)],
            [pltpu.VMEM((EB,), jnp.int32) for _ in range(2)],
            [pltpu.VMEM((EB,), jnp.int32) for _ in range(2)],
            [pltpu.VMEM((2 * EB, F), jnp.float32) for _ in range(2)],
            [pltpu.VMEM((EB, F), jnp.float32) for _ in range(2)],
            pltpu.VMEM_SHARED((np_, F), jnp.float32),
            [pltpu.SemaphoreType.DMA for _ in range(2)],
            [pltpu.SemaphoreType.DMA for _ in range(2)],
            [pltpu.SemaphoreType.DMA for _ in range(2)],
        ],
    )
    def ek(kqv_hbm, src_hbm, dst_hbm, zrows_hbm, out_hbm,
           sd, ikq, iv, db, kqb, vb, agg, semi, semg, sems):
        cid = lax.axis_index("c")
        sid = lax.axis_index("s")
        wid = sid * NCORE + cid
        base = wid * ew
        row0 = sid * rps
        rows = pl.ds(row0, rps)

        def stage_i(b, p):
            e0 = base + jnp.minimum(b, nb - 1) * EB
            pltpu.async_copy(src_hbm.at[pl.ds(e0, EB)], sd[p].at[0], semi[p])
            pltpu.async_copy(dst_hbm.at[pl.ds(e0, EB)], sd[p].at[1], semi[p])

        def stage_g(b, p, koff, qoff, voff, first=False):
            e0 = base + b * EB
            pltpu.make_async_copy(src_hbm.at[pl.ds(e0, EB)], sd[p].at[0],
                                  semi[p]).wait()
            pltpu.make_async_copy(dst_hbm.at[pl.ds(e0, EB)], sd[p].at[1],
                                  semi[p]).wait()
            if not first:
                @pl.when(b >= 2)
                def _():
                    pltpu.make_async_copy(vb[p], agg.at[db[p]],
                                          sems[p]).wait()
            for g in range(EB // 16):
                gd = pl.ds(g * 16, 16)
                s16 = sd[p][0, gd]
                d16 = sd[p][1, gd]
                ikq[p][gd] = d16 + koff
                ikq[p][pl.ds(EB + g * 16, 16)] = s16 + qoff
                iv[p][gd] = s16 + voff
                db[p][gd] = d16
            pltpu.async_copy(kqv_hbm.at[ikq[p]], kqb[p], semg[p])
            pltpu.async_copy(kqv_hbm.at[iv[p]], vb[p], semg[p])

        def finish(p):
            pltpu.make_async_copy(kqv_hbm.at[ikq[p]], kqb[p], semg[p]).wait()
            pltpu.make_async_copy(kqv_hbm.at[iv[p]], vb[p], semg[p]).wait()

            def edge(i, c):
                for g in range(F // 16):
                    sl = pl.ds(g * 16, 16)
                    d = jnp.exp(kqb[p][i, sl] + kqb[p][EB + i, sl]) + 1.0
                    vb[p][i, sl] = vb[p][i, sl] / d
                return c

            lax.fori_loop(0, EB, edge, 0)
            pltpu.async_copy(vb[p], agg.at[db[p]], sems[p], add=True)

        for ck in range(nc):
            koff = ck * n
            qoff = (nc + ck) * n
            voff = (2 * nc + ck) * n

            pltpu.sync_copy(zrows_hbm, agg.at[rows])
            plsc.subcore_barrier()

            stage_i(0, 0)
            stage_i(1, 1)
            stage_g(0, 0, koff, qoff, voff, first=True)

            def pair(j, carry):
                b0 = 2 * j
                stage_i(b0 + 2, 0)
                stage_g(b0 + 1, 1, koff, qoff, voff)
                finish(0)
                stage_i(b0 + 3, 1)
                stage_g(b0 + 2, 0, koff, qoff, voff)
                finish(1)
                return carry

            lax.fori_loop(0, (nb - 1) // 2, pair, 0)
            finish(0)
            pltpu.make_async_copy(src_hbm.at[pl.ds(base, EB)],
                                  sd[1].at[0], semi[1]).wait()
            pltpu.make_async_copy(dst_hbm.at[pl.ds(base, EB)],
                                  sd[1].at[1], semi[1]).wait()
            pltpu.make_async_copy(vb[1], agg.at[db[1]], sems[1]).wait()
            pltpu.make_async_copy(vb[0], agg.at[db[0]], sems[0]).wait()
            plsc.subcore_barrier()

            pltpu.sync_copy(agg.at[rows], out_hbm.at[cid, ck].at[rows])
            plsc.subcore_barrier()

    return ek


@functools.lru_cache(maxsize=None)
def _post_fn(nc, n, np_):
    def body(p_ref, kqv_ref, g_ref, b_ref, o_ref):
        h = jnp.maximum(p_ref[0, :n] + p_ref[1, :n] + kqv_ref[...], 0.0)
        m = jnp.mean(h, axis=0, keepdims=True)
        d = h - m
        var = jnp.mean(d * d, axis=0, keepdims=True)
        o_ref[...] = g_ref[...] * d / jnp.sqrt(var + 1e-5) + b_ref[...]

    return pl.pallas_call(
        body,
        grid=(nc,),
        in_specs=[
            pl.BlockSpec((NCORE, None, np_, F), lambda j: (0, j, 0, 0)),
            pl.BlockSpec((None, n, F), lambda j: (3 * nc + j, 0, 0)),
            pl.BlockSpec((None, 1, F), lambda j: (j, 0, 0)),
            pl.BlockSpec((None, 1, F), lambda j: (j, 0, 0)),
        ],
        out_specs=pl.BlockSpec((n, F), lambda j: (0, j)),
        out_shape=jax.ShapeDtypeStruct((n, nc * F), jnp.float32),
    )


@functools.lru_cache(maxsize=None)
def _pool_fn(n):
    br = 1000
    nr = n // br

    def body(b_ref, h_ref, o_ref):
        @pl.when(pl.program_id(0) == 0)
        def _():
            o_ref[...] = jnp.zeros_like(o_ref)

        oh = (b_ref[...] == lax.broadcasted_iota(jnp.int32, (1, NGRAPH), 1)
              ).astype(jnp.float32)
        o_ref[...] += lax.dot_general(
            oh, h_ref[...], (((0,), (0,)), ((), ())),
            precision=lax.Precision.HIGHEST,
            preferred_element_type=jnp.float32)

    return pl.pallas_call(
        body,
        grid=(nr,),
        in_specs=[
            pl.BlockSpec((br, 1), lambda i: (i, 0)),
            pl.BlockSpec((br, F), lambda i: (i, 0)),
        ],
        out_specs=pl.BlockSpec((NGRAPH, F), lambda i: (0, 0)),
        out_shape=jax.ShapeDtypeStruct((NGRAPH, F), jnp.float32),
    )


def kernel(x, edge_index, batch, params):
    n = x.shape[0]
    e = edge_index.shape[1]
    _, _, pad = _edge_geometry(e)
    src = jnp.concatenate([edge_index[0], jnp.zeros((pad,), jnp.int32)])
    dst = jnp.concatenate([edge_index[1], jnp.full((pad,), n, jnp.int32)])
    np_ = -(-n // 128) * 128
    zrows = jnp.zeros((np_ // NSUB, F), jnp.float32)

    h = x
    for i, p in enumerate(params['convs']):
        din, dout = p['Wk'].shape
        nc = dout // F
        wall = jnp.concatenate([-p['Wk'], -p['Wq'], p['Wv'], p['Ws']], axis=1)
        ball = jnp.concatenate([-p['bk'], -p['bq'], p['bv'], p['bs']])[None, :]
        kqv = _dense_fn(n, din, 4 * nc)(h, wall, ball)
        parts = _edge_fn(nc, n, e)(kqv.reshape(4 * nc * n, F), src, dst, zrows)
        g2 = params['bn'][i]['g'].reshape(nc, 1, F)
        b2 = params['bn'][i]['b'].reshape(nc, 1, F)
        h = _post_fn(nc, n, parts.shape[2])(parts, kqv, g2, b2)

    return _pool_fn(n)(batch.reshape(n, 1), h)
